# baseline (speedup 1.0000x reference)
import jax
import jax.numpy as jnp
from jax.experimental import pallas as pl
from jax.experimental.pallas import tpu as pltpu

G = 4
HO = 48
NT = 3
TC = 16
KH = 7
KLANE = 128
NPF = 12
COUT = 64


def _shift_rows(a, d, block):
    rows, cols = a.shape
    ri = jax.lax.broadcasted_iota(jnp.int32, a.shape, 0)
    if d > 0:
        s = jnp.concatenate([jnp.zeros((d, cols), a.dtype), a[: rows - d]], axis=0)
        return jnp.where((ri % block) < d, 0.0, s)
    dd = -d
    s = jnp.concatenate([a[dd:], jnp.zeros((dd, cols), a.dtype)], axis=0)
    return jnp.where((ri % block) >= block - dd, 0.0, s)


def _shift_lanes(a, lanes):
    rows, cols = a.shape
    if lanes > 0:
        return jnp.concatenate(
            [jnp.zeros((rows, lanes), a.dtype), a[:, : cols - lanes]], axis=1)
    ll = -lanes
    return jnp.concatenate(
        [a[:, ll:], jnp.zeros((rows, ll), a.dtype)], axis=1)


def _fused_kernel(x_ref, w_ref, b_ref, o_ref, rm_ref):
    for t in range(NT):
        lhs = x_ref[t].reshape(G * HO, KH * KLANE)
        acc = jnp.dot(lhs, w_ref[...], preferred_element_type=jnp.float32)
        z = jnp.maximum(acc + b_ref[...], 0.0)
        zr = z.reshape(G * 24, 2, TC * COUT)
        ev, od = zr[:, 0, :], zr[:, 1, :]
        r1 = jnp.maximum(jnp.maximum(ev, od), _shift_rows(od, 1, 24))
        r1r = r1.reshape(G * NPF, 2, TC * COUT)
        e2, o2 = r1r[:, 0, :], r1r[:, 1, :]
        r2 = jnp.maximum(
            jnp.maximum(jnp.maximum(e2, o2), _shift_rows(e2, 1, NPF)),
            jnp.maximum(_shift_rows(o2, 1, NPF), _shift_rows(e2, -1, NPF)))
        rm_ref[:, t * TC * COUT:(t + 1) * TC * COUT] = r2
    r = rm_ref[...]
    s = jnp.maximum(jnp.maximum(_shift_lanes(r, COUT), r), _shift_lanes(r, -COUT))
    tt = jnp.maximum(jnp.maximum(_shift_lanes(s, 2 * COUT), s),
                     _shift_lanes(s, -2 * COUT))
    u = jnp.maximum(_shift_lanes(tt, 2 * COUT), _shift_lanes(tt, -2 * COUT))
    o_ref[...] = jnp.concatenate(
        [u[:, 4 * m * COUT:(4 * m + 1) * COUT] for m in range(NPF)], axis=1)


def _prep(x_nchw):
    n = x_nchw.shape[0]
    x = jnp.transpose(x_nchw, (0, 2, 3, 1)).astype(jnp.float32)
    xp = jnp.pad(x, ((0, 0), (3, 3), (3, 7), (0, 0)))
    slabs = []
    for t in range(NT):
        for kh in range(KH):
            slabs.append(xp[:, kh:kh + 2 * HO:2, 32 * t:32 * t + 42, :])
    arr = jnp.stack(slabs, axis=0).reshape(NT, KH, n, HO, 126)
    arr = jnp.pad(arr, ((0, 0), (0, 0), (0, 0), (0, 0), (0, KLANE - 126)))
    arr = arr.transpose(0, 2, 3, 1, 4).reshape(NT, n, HO, KH * KLANE)
    return arr.astype(jnp.bfloat16)


def kernel(x_nchw, w_banded, bias_tile):
    n = x_nchw.shape[0]
    xprep = _prep(x_nchw)
    w_cat = jnp.reshape(w_banded[:, :KLANE, :TC * COUT], (KH * KLANE, TC * COUT))
    b1 = bias_tile[:, :TC * COUT]

    out = pl.pallas_call(
        _fused_kernel,
        out_shape=jax.ShapeDtypeStruct((n * NPF, NPF * COUT), jnp.float32),
        grid_spec=pltpu.PrefetchScalarGridSpec(
            num_scalar_prefetch=0,
            grid=(n // G,),
            in_specs=[
                pl.BlockSpec((NT, G, HO, KH * KLANE), lambda b: (0, b, 0, 0)),
                pl.BlockSpec((KH * KLANE, TC * COUT), lambda b: (0, 0)),
                pl.BlockSpec((1, TC * COUT), lambda b: (0, 0)),
            ],
            out_specs=pl.BlockSpec((G * NPF, NPF * COUT), lambda b: (b, 0)),
            scratch_shapes=[pltpu.VMEM((G * NPF, NT * TC * COUT), jnp.float32)],
        ),
        compiler_params=pltpu.CompilerParams(
            dimension_semantics=("parallel",),
        ),
    )(xprep, w_cat, b1)

    out = out.reshape(n, NPF, NPF, COUT).transpose(0, 3, 1, 2)
    return out.reshape(n, -1)


# speedup vs baseline: 2.4295x; 2.4295x over previous
import jax
import jax.numpy as jnp
from jax.experimental import pallas as pl
from jax.experimental.pallas import tpu as pltpu

G = 4
HO = 48
NT = 3
TC = 16
KH = 7
KLANE = 128
NPF = 12
COUT = 64


def _shift_rows(a, d, block):
    rows, cols = a.shape
    ri = jax.lax.broadcasted_iota(jnp.int32, a.shape, 0)
    if d > 0:
        s = jnp.concatenate([jnp.zeros((d, cols), a.dtype), a[: rows - d]], axis=0)
        return jnp.where((ri % block) < d, 0.0, s)
    dd = -d
    s = jnp.concatenate([a[dd:], jnp.zeros((dd, cols), a.dtype)], axis=0)
    return jnp.where((ri % block) >= block - dd, 0.0, s)


def _shift_lanes(a, lanes):
    rows, cols = a.shape
    if lanes > 0:
        return jnp.concatenate(
            [jnp.zeros((rows, lanes), a.dtype), a[:, : cols - lanes]], axis=1)
    ll = -lanes
    return jnp.concatenate(
        [a[:, ll:], jnp.zeros((rows, ll), a.dtype)], axis=1)


def _fused_kernel(x_ref, w_ref, b_ref, o_ref, rm_ref):
    for t in range(NT):
        lhs = x_ref[t].reshape(G * HO, KH * KLANE)
        acc = jnp.dot(lhs, w_ref[...], preferred_element_type=jnp.float32)
        z = jnp.maximum(acc + b_ref[...], 0.0)
        zr = z.reshape(G * 24, 2, TC * COUT)
        ev, od = zr[:, 0, :], zr[:, 1, :]
        r1 = jnp.maximum(jnp.maximum(ev, od), _shift_rows(od, 1, 24))
        r1r = r1.reshape(G * NPF, 2, TC * COUT)
        e2, o2 = r1r[:, 0, :], r1r[:, 1, :]
        r2 = jnp.maximum(
            jnp.maximum(jnp.maximum(e2, o2), _shift_rows(e2, 1, NPF)),
            jnp.maximum(_shift_rows(o2, 1, NPF), _shift_rows(e2, -1, NPF)))
        rm_ref[:, t * TC * COUT:(t + 1) * TC * COUT] = r2
    r = rm_ref[...]
    s = jnp.maximum(jnp.maximum(_shift_lanes(r, COUT), r), _shift_lanes(r, -COUT))
    tt = jnp.maximum(jnp.maximum(_shift_lanes(s, 2 * COUT), s),
                     _shift_lanes(s, -2 * COUT))
    u = jnp.maximum(_shift_lanes(tt, 2 * COUT), _shift_lanes(tt, -2 * COUT))
    o_ref[...] = jnp.concatenate(
        [u[:, 4 * m * COUT:(4 * m + 1) * COUT] for m in range(NPF)], axis=1)


def _prep(x_nchw):
    n = x_nchw.shape[0]
    x = jnp.transpose(x_nchw, (0, 2, 3, 1))
    xp = jnp.pad(x, ((0, 0), (3, 3), (3, 7), (0, 0)))
    xf = jnp.pad(xp.reshape(n, 102, 318), ((0, 0), (0, 0), (0, 2)))
    xf = xf.astype(jnp.bfloat16)
    phases = (xf[:, 0::2, :], xf[:, 1::2, :])
    tiles = []
    for t in range(NT):
        slabs = [phases[kh % 2][:, kh // 2:kh // 2 + HO, 96 * t:96 * t + KLANE]
                 for kh in range(KH)]
        tiles.append(jnp.concatenate(slabs, axis=-1))
    return jnp.stack(tiles, axis=0)


def kernel(x_nchw, w_banded, bias_tile):
    n = x_nchw.shape[0]
    xprep = _prep(x_nchw)
    w_cat = jnp.reshape(w_banded[:, :KLANE, :TC * COUT], (KH * KLANE, TC * COUT))
    b1 = bias_tile[:, :TC * COUT]

    out = pl.pallas_call(
        _fused_kernel,
        out_shape=jax.ShapeDtypeStruct((n * NPF, NPF * COUT), jnp.float32),
        grid_spec=pltpu.PrefetchScalarGridSpec(
            num_scalar_prefetch=0,
            grid=(n // G,),
            in_specs=[
                pl.BlockSpec((NT, G, HO, KH * KLANE), lambda b: (0, b, 0, 0)),
                pl.BlockSpec((KH * KLANE, TC * COUT), lambda b: (0, 0)),
                pl.BlockSpec((1, TC * COUT), lambda b: (0, 0)),
            ],
            out_specs=pl.BlockSpec((G * NPF, NPF * COUT), lambda b: (b, 0)),
            scratch_shapes=[pltpu.VMEM((G * NPF, NT * TC * COUT), jnp.float32)],
        ),
        compiler_params=pltpu.CompilerParams(
            dimension_semantics=("parallel",),
        ),
    )(xprep, w_cat, b1)

    out = out.reshape(n, NPF, NPF, COUT).transpose(0, 3, 1, 2)
    return out.reshape(n, -1)


# speedup vs baseline: 2.8826x; 1.1865x over previous
import jax
import jax.numpy as jnp
from jax.experimental import pallas as pl
from jax.experimental.pallas import tpu as pltpu

G = 4
HO = 48
NT = 3
TC = 16
KH = 7
KLANE = 128
NPF = 12
COUT = 64


def _shift_rows(a, d, block):
    rows, cols = a.shape
    ri = jax.lax.broadcasted_iota(jnp.int32, a.shape, 0)
    if d > 0:
        s = jnp.concatenate([jnp.zeros((d, cols), a.dtype), a[: rows - d]], axis=0)
        return jnp.where((ri % block) < d, 0.0, s)
    dd = -d
    s = jnp.concatenate([a[dd:], jnp.zeros((dd, cols), a.dtype)], axis=0)
    return jnp.where((ri % block) >= block - dd, 0.0, s)


def _shift_lanes(a, lanes):
    rows, cols = a.shape
    if lanes > 0:
        return jnp.concatenate(
            [jnp.zeros((rows, lanes), a.dtype), a[:, : cols - lanes]], axis=1)
    ll = -lanes
    return jnp.concatenate(
        [a[:, ll:], jnp.zeros((rows, ll), a.dtype)], axis=1)


def _fused_kernel(x_ref, w_ref, b_ref, o_ref, rm_ref):
    for t in range(NT):
        slabs = [
            x_ref[:, kh % 2, kh // 2:kh // 2 + HO, 96 * t:96 * t + KLANE]
            .reshape(G * HO, KLANE)
            for kh in range(KH)
        ]
        lhs = jnp.concatenate(slabs, axis=-1)
        acc = jnp.dot(lhs, w_ref[...], preferred_element_type=jnp.float32)
        z = jnp.maximum(acc + b_ref[...], 0.0)
        zr = z.reshape(G * 24, 2, TC * COUT)
        ev, od = zr[:, 0, :], zr[:, 1, :]
        r1 = jnp.maximum(jnp.maximum(ev, od), _shift_rows(od, 1, 24))
        r1r = r1.reshape(G * NPF, 2, TC * COUT)
        e2, o2 = r1r[:, 0, :], r1r[:, 1, :]
        r2 = jnp.maximum(
            jnp.maximum(jnp.maximum(e2, o2), _shift_rows(e2, 1, NPF)),
            jnp.maximum(_shift_rows(o2, 1, NPF), _shift_rows(e2, -1, NPF)))
        rm_ref[:, t * TC * COUT:(t + 1) * TC * COUT] = r2
    r = rm_ref[...]
    s = jnp.maximum(jnp.maximum(_shift_lanes(r, COUT), r), _shift_lanes(r, -COUT))
    tt = jnp.maximum(jnp.maximum(_shift_lanes(s, 2 * COUT), s),
                     _shift_lanes(s, -2 * COUT))
    u = jnp.maximum(_shift_lanes(tt, 2 * COUT), _shift_lanes(tt, -2 * COUT))
    o_ref[...] = jnp.concatenate(
        [u[:, 4 * m * COUT:(4 * m + 1) * COUT] for m in range(NPF)], axis=1)


def _prep(x_nchw):
    n = x_nchw.shape[0]
    x = jnp.transpose(x_nchw, (0, 2, 3, 1))
    xp = jnp.pad(x, ((0, 0), (3, 3), (3, 7), (0, 0)))
    xf = jnp.pad(xp.reshape(n, 102, 318), ((0, 0), (0, 0), (0, 2)))
    xf = xf.astype(jnp.bfloat16)
    return jnp.stack((xf[:, 0::2, :], xf[:, 1::2, :]), axis=1)


def kernel(x_nchw, w_banded, bias_tile):
    n = x_nchw.shape[0]
    xprep = _prep(x_nchw)
    w_cat = jnp.reshape(w_banded[:, :KLANE, :TC * COUT], (KH * KLANE, TC * COUT))
    b1 = bias_tile[:, :TC * COUT]

    out = pl.pallas_call(
        _fused_kernel,
        out_shape=jax.ShapeDtypeStruct((n * NPF, NPF * COUT), jnp.float32),
        grid_spec=pltpu.PrefetchScalarGridSpec(
            num_scalar_prefetch=0,
            grid=(n // G,),
            in_specs=[
                pl.BlockSpec((G, 2, 51, 320), lambda b: (b, 0, 0, 0)),
                pl.BlockSpec((KH * KLANE, TC * COUT), lambda b: (0, 0)),
                pl.BlockSpec((1, TC * COUT), lambda b: (0, 0)),
            ],
            out_specs=pl.BlockSpec((G * NPF, NPF * COUT), lambda b: (b, 0)),
            scratch_shapes=[pltpu.VMEM((G * NPF, NT * TC * COUT), jnp.float32)],
        ),
        compiler_params=pltpu.CompilerParams(
            dimension_semantics=("parallel",),
        ),
    )(xprep, w_cat, b1)

    out = out.reshape(n, NPF, NPF, COUT).transpose(0, 3, 1, 2)
    return out.reshape(n, -1)


# speedup vs baseline: 3.4897x; 1.2106x over previous
import jax
import jax.numpy as jnp
from jax.experimental import pallas as pl
from jax.experimental.pallas import tpu as pltpu

G = 4
HO = 48
NT = 3
TC = 16
KH = 7
KLANE = 128
NPF = 12
COUT = 64


def _shift_rows(a, d, block):
    rows, cols = a.shape
    ri = jax.lax.broadcasted_iota(jnp.int32, a.shape, 0)
    if d > 0:
        s = jnp.concatenate([jnp.zeros((d, cols), a.dtype), a[: rows - d]], axis=0)
        return jnp.where((ri % block) < d, 0.0, s)
    dd = -d
    s = jnp.concatenate([a[dd:], jnp.zeros((dd, cols), a.dtype)], axis=0)
    return jnp.where((ri % block) >= block - dd, 0.0, s)


def _shift_lanes(a, lanes):
    rows, cols = a.shape
    if lanes > 0:
        return jnp.concatenate(
            [jnp.zeros((rows, lanes), a.dtype), a[:, : cols - lanes]], axis=1)
    ll = -lanes
    return jnp.concatenate(
        [a[:, ll:], jnp.zeros((rows, ll), a.dtype)], axis=1)


def _fused_kernel(x_ref, w_ref, b_ref, p_ref, o_ref, rm_ref):
    for t in range(NT):
        slabs = [
            x_ref[:, kh % 2, kh // 2:kh // 2 + HO, 96 * t:96 * t + KLANE]
            .reshape(G * HO, KLANE)
            for kh in range(KH)
        ]
        lhs = jnp.concatenate(slabs, axis=-1)
        acc = jnp.dot(lhs, w_ref[...], preferred_element_type=jnp.float32)
        z = jnp.maximum(acc + b_ref[...], 0.0)
        zr = z.reshape(G * 24, 2, TC * COUT)
        ev, od = zr[:, 0, :], zr[:, 1, :]
        r1 = jnp.maximum(jnp.maximum(ev, od), _shift_rows(od, 1, 24))
        r1r = r1.reshape(G * NPF, 2, TC * COUT)
        e2, o2 = r1r[:, 0, :], r1r[:, 1, :]
        r2 = jnp.maximum(
            jnp.maximum(jnp.maximum(e2, o2), _shift_rows(e2, 1, NPF)),
            jnp.maximum(_shift_rows(o2, 1, NPF), _shift_rows(e2, -1, NPF)))
        rm_ref[:, t * TC * COUT:(t + 1) * TC * COUT] = r2
    r = rm_ref[...]
    s = jnp.maximum(jnp.maximum(_shift_lanes(r, COUT), r), _shift_lanes(r, -COUT))
    tg = []
    for m2 in range(NPF):
        t = jnp.maximum(s[:, 256 * m2:256 * m2 + COUT],
                        s[:, 256 * m2 + 128:256 * m2 + 128 + COUT])
        if 256 * m2 + 256 < NT * TC * COUT:
            t = jnp.maximum(t, s[:, 256 * m2 + 256:256 * m2 + 256 + COUT])
        tg.append(t)
    pieces = [tg[0]] + [jnp.maximum(tg[m2 - 1], tg[m2]) for m2 in range(1, NPF)]
    for g in range(G):
        p3 = jnp.concatenate(
            [pieces[m2][g * NPF:(g + 1) * NPF, :] for m2 in range(NPF)], axis=0)
        hi = p3.astype(jnp.bfloat16)
        lo = (p3 - hi.astype(jnp.float32)).astype(jnp.bfloat16)
        dn = (((0,), (0,)), ((), ()))
        og = (jax.lax.dot_general(hi, p_ref[...], dn,
                                  preferred_element_type=jnp.float32)
              + jax.lax.dot_general(lo, p_ref[...], dn,
                                    preferred_element_type=jnp.float32))
        o_ref[g * COUT:(g + 1) * COUT, :] = og


def _prep(x_nchw):
    n = x_nchw.shape[0]
    x = jnp.transpose(x_nchw, (0, 2, 3, 1))
    xp = jnp.pad(x, ((0, 0), (3, 3), (3, 7), (0, 0)))
    xf = jnp.pad(xp.reshape(n, 102, 318), ((0, 0), (0, 0), (0, 2)))
    xf = xf.astype(jnp.bfloat16)
    return jnp.stack((xf[:, 0::2, :], xf[:, 1::2, :]), axis=1)


def kernel(x_nchw, w_banded, bias_tile):
    n = x_nchw.shape[0]
    xprep = _prep(x_nchw)
    w_cat = jnp.reshape(w_banded[:, :KLANE, :TC * COUT], (KH * KLANE, TC * COUT))
    b1 = bias_tile[:, :TC * COUT]
    ri = jax.lax.broadcasted_iota(jnp.int32, (NPF * NPF, NPF * NPF), 0)
    qi = jax.lax.broadcasted_iota(jnp.int32, (NPF * NPF, NPF * NPF), 1)
    perm = ((ri % NPF) * NPF + ri // NPF == qi).astype(jnp.bfloat16)

    out = pl.pallas_call(
        _fused_kernel,
        out_shape=jax.ShapeDtypeStruct((n * COUT, NPF * NPF), jnp.float32),
        grid_spec=pltpu.PrefetchScalarGridSpec(
            num_scalar_prefetch=0,
            grid=(n // G,),
            in_specs=[
                pl.BlockSpec((G, 2, 51, 320), lambda b: (b, 0, 0, 0)),
                pl.BlockSpec((KH * KLANE, TC * COUT), lambda b: (0, 0)),
                pl.BlockSpec((1, TC * COUT), lambda b: (0, 0)),
                pl.BlockSpec((NPF * NPF, NPF * NPF), lambda b: (0, 0)),
            ],
            out_specs=pl.BlockSpec((G * COUT, NPF * NPF), lambda b: (b, 0)),
            scratch_shapes=[pltpu.VMEM((G * NPF, NT * TC * COUT), jnp.float32)],
        ),
        compiler_params=pltpu.CompilerParams(
            dimension_semantics=("parallel",),
        ),
    )(xprep, w_cat, b1, perm)

    return out.reshape(n, -1)


# speedup vs baseline: 3.5892x; 1.0285x over previous
import jax
import jax.numpy as jnp
from jax.experimental import pallas as pl
from jax.experimental.pallas import tpu as pltpu

G = 8
HO = 48
NT = 3
TC = 16
KH = 7
KLANE = 128
NPF = 12
COUT = 64


def _shift_rows(a, d, block):
    rows, cols = a.shape
    ri = jax.lax.broadcasted_iota(jnp.int32, a.shape, 0)
    if d > 0:
        s = jnp.concatenate([jnp.zeros((d, cols), a.dtype), a[: rows - d]], axis=0)
        return jnp.where((ri % block) < d, 0.0, s)
    dd = -d
    s = jnp.concatenate([a[dd:], jnp.zeros((dd, cols), a.dtype)], axis=0)
    return jnp.where((ri % block) >= block - dd, 0.0, s)


def _shift_lanes(a, lanes):
    rows, cols = a.shape
    if lanes > 0:
        return jnp.concatenate(
            [jnp.zeros((rows, lanes), a.dtype), a[:, : cols - lanes]], axis=1)
    ll = -lanes
    return jnp.concatenate(
        [a[:, ll:], jnp.zeros((rows, ll), a.dtype)], axis=1)


def _fused_kernel(x_ref, w_ref, b_ref, p_ref, o_ref, rm_ref):
    for t in range(NT):
        slabs = [
            x_ref[:, kh % 2, kh // 2:kh // 2 + HO, 96 * t:96 * t + KLANE]
            .reshape(G * HO, KLANE)
            for kh in range(KH)
        ]
        lhs = jnp.concatenate(slabs, axis=-1)
        acc = jnp.dot(lhs, w_ref[...], preferred_element_type=jnp.float32)
        z = jnp.maximum(acc + b_ref[...], 0.0)
        zr = z.reshape(G * 24, 2, TC * COUT)
        ev, od = zr[:, 0, :], zr[:, 1, :]
        r1 = jnp.maximum(jnp.maximum(ev, od), _shift_rows(od, 1, 24))
        r1r = r1.reshape(G * NPF, 2, TC * COUT)
        e2, o2 = r1r[:, 0, :], r1r[:, 1, :]
        r2 = jnp.maximum(
            jnp.maximum(jnp.maximum(e2, o2), _shift_rows(e2, 1, NPF)),
            jnp.maximum(_shift_rows(o2, 1, NPF), _shift_rows(e2, -1, NPF)))
        rm_ref[:, t * TC * COUT:(t + 1) * TC * COUT] = r2
    r = rm_ref[...]
    s = jnp.maximum(jnp.maximum(_shift_lanes(r, COUT), r), _shift_lanes(r, -COUT))
    tg = []
    for m2 in range(NPF):
        t = jnp.maximum(s[:, 256 * m2:256 * m2 + COUT],
                        s[:, 256 * m2 + 128:256 * m2 + 128 + COUT])
        if 256 * m2 + 256 < NT * TC * COUT:
            t = jnp.maximum(t, s[:, 256 * m2 + 256:256 * m2 + 256 + COUT])
        tg.append(t)
    pieces = [tg[0]] + [jnp.maximum(tg[m2 - 1], tg[m2]) for m2 in range(1, NPF)]
    for g in range(G):
        p3 = jnp.concatenate(
            [pieces[m2][g * NPF:(g + 1) * NPF, :] for m2 in range(NPF)], axis=0)
        hi = p3.astype(jnp.bfloat16)
        lo = (p3 - hi.astype(jnp.float32)).astype(jnp.bfloat16)
        dn = (((0,), (0,)), ((), ()))
        og = (jax.lax.dot_general(hi, p_ref[...], dn,
                                  preferred_element_type=jnp.float32)
              + jax.lax.dot_general(lo, p_ref[...], dn,
                                    preferred_element_type=jnp.float32))
        o_ref[g * COUT:(g + 1) * COUT, :] = og


def _prep(x_nchw):
    n = x_nchw.shape[0]
    x = jnp.transpose(x_nchw, (0, 2, 3, 1))
    xp = jnp.pad(x, ((0, 0), (3, 3), (3, 7), (0, 0)))
    xf = jnp.pad(xp.reshape(n, 102, 318), ((0, 0), (0, 0), (0, 2)))
    xf = xf.astype(jnp.bfloat16)
    return jnp.stack((xf[:, 0::2, :], xf[:, 1::2, :]), axis=1)


def kernel(x_nchw, w_banded, bias_tile):
    n = x_nchw.shape[0]
    xprep = _prep(x_nchw)
    w_cat = jnp.reshape(w_banded[:, :KLANE, :TC * COUT], (KH * KLANE, TC * COUT))
    b1 = bias_tile[:, :TC * COUT]
    ri = jax.lax.broadcasted_iota(jnp.int32, (NPF * NPF, NPF * NPF), 0)
    qi = jax.lax.broadcasted_iota(jnp.int32, (NPF * NPF, NPF * NPF), 1)
    perm = ((ri % NPF) * NPF + ri // NPF == qi).astype(jnp.bfloat16)

    out = pl.pallas_call(
        _fused_kernel,
        out_shape=jax.ShapeDtypeStruct((n * COUT, NPF * NPF), jnp.float32),
        grid_spec=pltpu.PrefetchScalarGridSpec(
            num_scalar_prefetch=0,
            grid=(n // G,),
            in_specs=[
                pl.BlockSpec((G, 2, 51, 320), lambda b: (b, 0, 0, 0)),
                pl.BlockSpec((KH * KLANE, TC * COUT), lambda b: (0, 0)),
                pl.BlockSpec((1, TC * COUT), lambda b: (0, 0)),
                pl.BlockSpec((NPF * NPF, NPF * NPF), lambda b: (0, 0)),
            ],
            out_specs=pl.BlockSpec((G * COUT, NPF * NPF), lambda b: (b, 0)),
            scratch_shapes=[pltpu.VMEM((G * NPF, NT * TC * COUT), jnp.float32)],
        ),
        compiler_params=pltpu.CompilerParams(
            dimension_semantics=("parallel",),
        ),
    )(xprep, w_cat, b1, perm)

    return out.reshape(n, -1)


# speedup vs baseline: 4.9507x; 1.3793x over previous
import jax
import jax.numpy as jnp
from jax.experimental import pallas as pl
from jax.experimental.pallas import tpu as pltpu

G = 8
HO = 48
NT = 3
TC = 16
KH = 7
KLANE = 128
NPF = 12
COUT = 64


NEG = float("-inf")


def _shift_rows(a, d, block, valid=None):
    rows, cols = a.shape
    ri = jax.lax.broadcasted_iota(jnp.int32, a.shape, 0)
    if d > 0:
        s = jnp.concatenate([jnp.full((d, cols), NEG, a.dtype), a[: rows - d]],
                            axis=0)
        return jnp.where((ri % block) < d, NEG, s)
    dd = -d
    s = jnp.concatenate([a[dd:], jnp.full((dd, cols), NEG, a.dtype)], axis=0)
    return jnp.where((ri % block) >= (valid or block) - dd, NEG, s)


def _shift_lanes(a, lanes):
    rows, cols = a.shape
    if lanes > 0:
        return jnp.concatenate(
            [jnp.full((rows, lanes), NEG, a.dtype), a[:, : cols - lanes]], axis=1)
    ll = -lanes
    return jnp.concatenate(
        [a[:, ll:], jnp.full((rows, ll), NEG, a.dtype)], axis=1)


def _fused_kernel(x_ref, w_ref, b_ref, p_ref, o_ref, rm_ref):
    gb = G * 16
    for t in range(NT):
        lhs = jnp.concatenate([
            jnp.concatenate([
                x_ref[:, 2 * j + kh, :, 96 * t:96 * t + KLANE]
                .reshape(gb, KLANE)
                for kh in range(KH)
            ], axis=-1)
            for j in range(4)
        ], axis=0)
        z = jnp.dot(lhs, w_ref[...], preferred_element_type=jnp.float32)
        e0, e1, e2, e3 = (z[j * gb:(j + 1) * gb] for j in range(4))
        p = jnp.maximum(jnp.maximum(e0, e1), jnp.maximum(e2, e3))
        r2 = jnp.maximum(
            jnp.maximum(jnp.maximum(p, _shift_rows(p, 1, 16)),
                        _shift_rows(e3, 2, 16)),
            jnp.maximum(_shift_rows(e0, -1, 16, NPF),
                        _shift_rows(e1, -1, 16, NPF)))
        rm_ref[:, t * TC * COUT:(t + 1) * TC * COUT] = r2
    r = rm_ref[...]
    s = jnp.maximum(jnp.maximum(_shift_lanes(r, COUT), r), _shift_lanes(r, -COUT))
    tg = []
    for m2 in range(NPF):
        t = jnp.maximum(s[:, 256 * m2:256 * m2 + COUT],
                        s[:, 256 * m2 + 128:256 * m2 + 128 + COUT])
        if 256 * m2 + 256 < NT * TC * COUT:
            t = jnp.maximum(t, s[:, 256 * m2 + 256:256 * m2 + 256 + COUT])
        tg.append(t)
    pieces = [tg[0]] + [jnp.maximum(tg[m2 - 1], tg[m2]) for m2 in range(1, NPF)]
    for g in range(G):
        p3 = jnp.concatenate(
            [pieces[m2][g * 16:g * 16 + NPF, :] for m2 in range(NPF)], axis=0)
        p3 = jnp.maximum(p3 + b_ref[...], 0.0)
        hi = p3.astype(jnp.bfloat16)
        lo = (p3 - hi.astype(jnp.float32)).astype(jnp.bfloat16)
        dn = (((0,), (0,)), ((), ()))
        og = (jax.lax.dot_general(hi, p_ref[...], dn,
                                  preferred_element_type=jnp.float32)
              + jax.lax.dot_general(lo, p_ref[...], dn,
                                    preferred_element_type=jnp.float32))
        o_ref[g * COUT:(g + 1) * COUT, :] = og


def _prep(x_nchw):
    n = x_nchw.shape[0]
    x = jnp.transpose(x_nchw, (0, 2, 3, 1))
    xp = jnp.pad(x, ((0, 0), (3, 3), (3, 7), (0, 0)))
    xf = jnp.pad(xp.reshape(n, 102, 318), ((0, 0), (0, 26), (0, 2)))
    xf = xf.astype(jnp.bfloat16)
    base = xf.reshape(n, 16, 8, 320).transpose(0, 2, 1, 3)
    shifted = jnp.pad(base[:, :7, 1:, :], ((0, 0), (0, 0), (0, 1), (0, 0)))
    return jnp.concatenate([base, shifted], axis=1)


def kernel(x_nchw, w_banded, bias_tile):
    n = x_nchw.shape[0]
    xprep = _prep(x_nchw)
    w_cat = jnp.reshape(w_banded[:, :KLANE, :TC * COUT], (KH * KLANE, TC * COUT))
    b1 = bias_tile[:, :COUT]
    ri = jax.lax.broadcasted_iota(jnp.int32, (NPF * NPF, NPF * NPF), 0)
    qi = jax.lax.broadcasted_iota(jnp.int32, (NPF * NPF, NPF * NPF), 1)
    perm = ((ri % NPF) * NPF + ri // NPF == qi).astype(jnp.bfloat16)

    out = pl.pallas_call(
        _fused_kernel,
        out_shape=jax.ShapeDtypeStruct((n * COUT, NPF * NPF), jnp.float32),
        grid_spec=pltpu.PrefetchScalarGridSpec(
            num_scalar_prefetch=0,
            grid=(n // G,),
            in_specs=[
                pl.BlockSpec((G, 15, 16, 320), lambda b: (b, 0, 0, 0)),
                pl.BlockSpec((KH * KLANE, TC * COUT), lambda b: (0, 0)),
                pl.BlockSpec((1, COUT), lambda b: (0, 0)),
                pl.BlockSpec((NPF * NPF, NPF * NPF), lambda b: (0, 0)),
            ],
            out_specs=pl.BlockSpec((G * COUT, NPF * NPF), lambda b: (b, 0)),
            scratch_shapes=[pltpu.VMEM((G * 16, NT * TC * COUT), jnp.float32)],
        ),
        compiler_params=pltpu.CompilerParams(
            dimension_semantics=("parallel",),
        ),
    )(xprep, w_cat, b1, perm)

    return out.reshape(n, -1)
